# baseline (speedup 1.0000x reference)
import jax
import jax.numpy as jnp
from jax.experimental import pallas as pl
from jax.experimental.pallas import tpu as pltpu

_INV_T = 100.0
_L, _P, _D = 16, 768, 256
_N = _L * _P
_BI = 128
_JC = 1024


def _pack_body(lm_ref, pf_ref, mm_ref, u_ref, cnt_ref):
    sims = jax.lax.dot_general(
        lm_ref[...], pf_ref[...],
        (((1,), (1,)), ((), ())),
        preferred_element_type=jnp.float32,
    )
    mm = mm_ref[...].astype(jnp.int32)
    cls = ((mm >> 1) & 1) + (mm & 1)
    q = jnp.round((sims + 1.0) * 2048.0).astype(jnp.int32)
    u_ref[...] = (-(cls * 8192 + q)).astype(jnp.int16)
    pos = jnp.where(mm >= 2, 1.0, 0.0)
    pn = jnp.where((mm & 1) == 1, 1.0, 0.0)
    lane = jax.lax.broadcasted_iota(jnp.int32, (1, 128), 1)
    kpos = jnp.sum(pos).astype(jnp.int32)
    kpn = jnp.sum(pn).astype(jnp.int32)
    cnt_ref[...] = jnp.where(lane == 0, kpos, jnp.where(lane == 1, kpn, 0))


def _main_body(k_ref, u_row_ref, out_ref):
    nb = (k_ref[0, 0] + (_BI - 1)) // _BI
    kpos_f = k_ref[0, 0].astype(jnp.float32)
    kpn_f = k_ref[0, 1].astype(jnp.float32)

    def body(blk, carry):
        num_acc, npos_acc = carry
        off = pl.multiple_of(blk * _BI, _BI)
        k_row = -(u_row_ref[:, pl.ds(off, _BI)].astype(jnp.int32))
        cls_r = k_row >> 13
        s_i = (k_row & 8191).astype(jnp.float32) * (1.0 / 2048.0) - 1.0
        three = jnp.concatenate(
            [s_i,
             jnp.where(cls_r == 2, 1.0, 0.0),
             jnp.where(cls_r >= 1, 1.0, 0.0)], axis=0)
        three_t = jax.lax.transpose(three, (1, 0))
        s_col = three_t[:, 0:1]
        pos_col = three_t[:, 1:2]
        pn_col = three_t[:, 2:3]
        sc50 = (s_col * (0.5 * _INV_T)).astype(jnp.bfloat16)

        accs = []
        for jc in range(_N // _JC):
            k = -(u_row_ref[:, jc * _JC:(jc + 1) * _JC].astype(jnp.int32))
            cls = k >> 13
            m_pn = cls >= 1
            m_pp = cls == 2
            s_j = ((k & 8191).astype(jnp.float32)
                   * (1.0 / 2048.0) - 1.0)
            sj50 = (s_j * (0.5 * _INV_T)).astype(jnp.bfloat16)
            t = jax.lax.tanh(sj50 - sc50)
            w = jnp.concatenate(
                [jnp.where(m_pn, 1.0, 0.0),
                 jnp.where(m_pp, 1.0, 0.0)],
                axis=0).astype(jnp.bfloat16)
            accs.append(jax.lax.dot_general(
                t, w, (((1,), (1,)), ((), ())),
                preferred_element_type=jnp.float32,
            ))
        while len(accs) > 1:
            accs = [a + b for a, b in zip(accs[::2], accs[1::2])]
        acc = accs[0]
        rpn = 1.0 + 0.5 * (kpn_f + acc[:, 0:1])
        rp = 1.0 + 0.5 * (kpos_f + acc[:, 1:2])
        num_acc = num_acc + jnp.sum(pos_col * rp / rpn)
        npos_acc = npos_acc + jnp.sum(pos_col)
        return num_acc, npos_acc

    num, npos = jax.lax.fori_loop(
        0, nb, body, (jnp.float32(0.0), jnp.float32(0.0)))
    lane = jax.lax.broadcasted_iota(jnp.int32, (1, 128), 1)
    out_ref[...] = jnp.where(lane == 0, -(num / npos), 0.0)


def kernel(landmark_embeddings, patch_features, pos_patches, pos_neg_patches):
    mm = ((pos_patches.astype(jnp.uint8) << 1)
          | pos_neg_patches.astype(jnp.uint8))

    u, cnt = pl.pallas_call(
        _pack_body,
        out_shape=(
            jax.ShapeDtypeStruct((_L, _P), jnp.int16),
            jax.ShapeDtypeStruct((1, 128), jnp.int32),
        ),
    )(landmark_embeddings, patch_features, mm)

    u_sorted = jax.lax.sort(u.reshape(-1), dimension=0).reshape(1, _N)

    out = pl.pallas_call(
        _main_body,
        in_specs=[
            pl.BlockSpec(memory_space=pltpu.SMEM),
            pl.BlockSpec((1, _N), lambda: (0, 0)),
        ],
        out_specs=pl.BlockSpec((1, 128), lambda: (0, 0)),
        out_shape=jax.ShapeDtypeStruct((1, 128), jnp.float32),
    )(cnt, u_sorted)

    return out[0, 0]


# speedup vs baseline: 1.0512x; 1.0512x over previous
import jax
import jax.numpy as jnp
from jax.experimental import pallas as pl
from jax.experimental.pallas import tpu as pltpu

_INV_T = 100.0
_L, _P, _D = 16, 768, 256
_N = _L * _P
_BI = 128
_JC = 1024


def _pack_body(lm_ref, pf_ref, mm_ref, u_ref, cnt_ref):
    sims = jax.lax.dot_general(
        lm_ref[...], pf_ref[...],
        (((1,), (1,)), ((), ())),
        preferred_element_type=jnp.float32,
    )
    mm = mm_ref[...].astype(jnp.int32)
    pos = jnp.where(mm >= 2, 1.0, 0.0)
    pn = jnp.where((mm & 1) == 1, 1.0, 0.0)
    u2d = -(sims + 3.0 * pn + 12.0 * pos)
    for r in range(_L):
        u_ref[:, r * _P:(r + 1) * _P] = u2d[r:r + 1, :]
    lane = jax.lax.broadcasted_iota(jnp.int32, (1, 128), 1)
    kpos = jnp.sum(pos).astype(jnp.int32)
    kpn = jnp.sum(pn).astype(jnp.int32)
    cnt_ref[...] = jnp.where(lane == 0, kpos, jnp.where(lane == 1, kpn, 0))


def _main_body(k_ref, u_row_ref, out_ref):
    nb = (k_ref[0, 0] + (_BI - 1)) // _BI
    kpos_f = k_ref[0, 0].astype(jnp.float32)
    kpn_f = k_ref[0, 1].astype(jnp.float32)

    def body(blk, carry):
        num_acc, npos_acc = carry
        off = pl.multiple_of(blk * _BI, _BI)
        u_row = u_row_ref[:, pl.ds(off, _BI)]
        u_col = jax.lax.transpose(u_row, (1, 0))
        pos_col = jnp.where(u_col < -10.0, 1.0, 0.0)
        pn_col = jnp.where(u_col < -1.5, 1.0, 0.0)
        s_col = -u_col - 3.0 * pn_col - 12.0 * pos_col
        sc50 = (s_col * (0.5 * _INV_T)).astype(jnp.bfloat16)

        accs = []
        for jc in range(_N // _JC):
            v = u_row_ref[:, jc * _JC:(jc + 1) * _JC]
            m_pn = v < -1.5
            m_pp = v < -10.0
            s_j = (-v - jnp.where(m_pn, 3.0, 0.0)
                   - jnp.where(m_pp, 12.0, 0.0))
            sj50 = (s_j * (0.5 * _INV_T)).astype(jnp.bfloat16)
            t = jax.lax.tanh(sj50 - sc50)
            w = jnp.concatenate(
                [jnp.where(m_pn, 1.0, 0.0),
                 jnp.where(m_pp, 1.0, 0.0)],
                axis=0).astype(jnp.bfloat16)
            accs.append(jax.lax.dot_general(
                t, w, (((1,), (1,)), ((), ())),
                preferred_element_type=jnp.float32,
            ))
        while len(accs) > 1:
            accs = [a + b for a, b in zip(accs[::2], accs[1::2])]
        acc = accs[0]
        rpn = 1.0 + 0.5 * (kpn_f + acc[:, 0:1])
        rp = 1.0 + 0.5 * (kpos_f + acc[:, 1:2])
        num_acc = num_acc + pos_col * rp / rpn
        npos_acc = npos_acc + pos_col
        return num_acc, npos_acc

    zero_col = jnp.zeros((_BI, 1), jnp.float32)
    num_v, npos_v = jax.lax.fori_loop(0, nb, body, (zero_col, zero_col))
    num = jnp.sum(num_v)
    npos = jnp.sum(npos_v)
    lane = jax.lax.broadcasted_iota(jnp.int32, (1, 128), 1)
    out_ref[...] = jnp.where(lane == 0, -(num / npos), 0.0)


def kernel(landmark_embeddings, patch_features, pos_patches, pos_neg_patches):
    mm = ((pos_patches.astype(jnp.uint8) << 1)
          | pos_neg_patches.astype(jnp.uint8))

    u, cnt = pl.pallas_call(
        _pack_body,
        out_shape=(
            jax.ShapeDtypeStruct((1, _N), jnp.float32),
            jax.ShapeDtypeStruct((1, 128), jnp.int32),
        ),
    )(landmark_embeddings, patch_features, mm)

    u_sorted = jax.lax.sort(u.reshape(-1), dimension=0).reshape(1, _N)

    out = pl.pallas_call(
        _main_body,
        in_specs=[
            pl.BlockSpec(memory_space=pltpu.SMEM),
            pl.BlockSpec((1, _N), lambda: (0, 0)),
        ],
        out_specs=pl.BlockSpec((1, 128), lambda: (0, 0)),
        out_shape=jax.ShapeDtypeStruct((1, 128), jnp.float32),
    )(cnt, u_sorted)

    return out[0, 0]


# speedup vs baseline: 1.1008x; 1.0472x over previous
import jax
import jax.numpy as jnp
from jax.experimental import pallas as pl
from jax.experimental.pallas import tpu as pltpu

_INV_T = 100.0
_L, _P, _D = 16, 768, 256
_N = _L * _P
_BI = 256
_JC = 1024


def _pack_body(lm_ref, pf_ref, mm_ref, u_ref, cnt_ref):
    sims = jax.lax.dot_general(
        lm_ref[...], pf_ref[...],
        (((1,), (1,)), ((), ())),
        preferred_element_type=jnp.float32,
    )
    mm = mm_ref[...].astype(jnp.int32)
    pos = jnp.where(mm >= 2, 1.0, 0.0)
    pn = jnp.where((mm & 1) == 1, 1.0, 0.0)
    u2d = -(sims + 3.0 * pn + 12.0 * pos)
    for r in range(_L):
        u_ref[:, r * _P:(r + 1) * _P] = u2d[r:r + 1, :]
    lane = jax.lax.broadcasted_iota(jnp.int32, (1, 128), 1)
    kpos = jnp.sum(pos).astype(jnp.int32)
    kpn = jnp.sum(pn).astype(jnp.int32)
    cnt_ref[...] = jnp.where(lane == 0, kpos, jnp.where(lane == 1, kpn, 0))


def _main_body(k_ref, u_row_ref, out_ref):
    nb = (k_ref[0, 0] + (_BI - 1)) // _BI
    kpos_f = k_ref[0, 0].astype(jnp.float32)
    kpn_f = k_ref[0, 1].astype(jnp.float32)

    def body(blk, carry):
        num_acc, npos_acc = carry
        off = pl.multiple_of(blk * _BI, _BI)
        u_row = u_row_ref[:, pl.ds(off, _BI)]
        u_col = jax.lax.transpose(u_row, (1, 0))
        pos_col = jnp.where(u_col < -10.0, 1.0, 0.0)
        pn_col = jnp.where(u_col < -1.5, 1.0, 0.0)
        s_col = -u_col - 3.0 * pn_col - 12.0 * pos_col
        sc50 = (s_col * (0.5 * _INV_T)).astype(jnp.bfloat16)

        accs = []
        for jc in range(_N // _JC):
            v = u_row_ref[:, jc * _JC:(jc + 1) * _JC]
            m_pn = v < -1.5
            m_pp = v < -10.0
            s_j = (-v - jnp.where(m_pn, 3.0, 0.0)
                   - jnp.where(m_pp, 12.0, 0.0))
            sj50 = (s_j * (0.5 * _INV_T)).astype(jnp.bfloat16)
            t = jax.lax.tanh(sj50 - sc50)
            w = jnp.concatenate(
                [jnp.where(m_pn, 1.0, 0.0),
                 jnp.where(m_pp, 1.0, 0.0)],
                axis=0).astype(jnp.bfloat16)
            accs.append(jax.lax.dot_general(
                t, w, (((1,), (1,)), ((), ())),
                preferred_element_type=jnp.float32,
            ))
        while len(accs) > 1:
            accs = [a + b for a, b in zip(accs[::2], accs[1::2])]
        acc = accs[0]
        rpn = 1.0 + 0.5 * (kpn_f + acc[:, 0:1])
        rp = 1.0 + 0.5 * (kpos_f + acc[:, 1:2])
        num_acc = num_acc + pos_col * rp / rpn
        npos_acc = npos_acc + pos_col
        return num_acc, npos_acc

    zero_col = jnp.zeros((_BI, 1), jnp.float32)
    num_v, npos_v = jax.lax.fori_loop(0, nb, body, (zero_col, zero_col))
    num = jnp.sum(num_v)
    npos = jnp.sum(npos_v)
    lane = jax.lax.broadcasted_iota(jnp.int32, (1, 128), 1)
    out_ref[...] = jnp.where(lane == 0, -(num / npos), 0.0)


def kernel(landmark_embeddings, patch_features, pos_patches, pos_neg_patches):
    mm = ((pos_patches.astype(jnp.uint8) << 1)
          | pos_neg_patches.astype(jnp.uint8))

    u, cnt = pl.pallas_call(
        _pack_body,
        out_shape=(
            jax.ShapeDtypeStruct((1, _N), jnp.float32),
            jax.ShapeDtypeStruct((1, 128), jnp.int32),
        ),
    )(landmark_embeddings, patch_features, mm)

    u_sorted = jax.lax.sort(u.reshape(-1), dimension=0).reshape(1, _N)

    out = pl.pallas_call(
        _main_body,
        in_specs=[
            pl.BlockSpec(memory_space=pltpu.SMEM),
            pl.BlockSpec((1, _N), lambda: (0, 0)),
        ],
        out_specs=pl.BlockSpec((1, 128), lambda: (0, 0)),
        out_shape=jax.ShapeDtypeStruct((1, 128), jnp.float32),
    )(cnt, u_sorted)

    return out[0, 0]


# speedup vs baseline: 1.1101x; 1.0084x over previous
import jax
import jax.numpy as jnp
from jax.experimental import pallas as pl
from jax.experimental.pallas import tpu as pltpu

_INV_T = 100.0
_L, _P, _D = 16, 768, 256
_N = _L * _P
_BI = 384
_JC = 1024


def _pack_body(lm_ref, pf_ref, mm_ref, u_ref, cnt_ref):
    sims = jax.lax.dot_general(
        lm_ref[...], pf_ref[...],
        (((1,), (1,)), ((), ())),
        preferred_element_type=jnp.float32,
    )
    mm = mm_ref[...].astype(jnp.int32)
    pos = jnp.where(mm >= 2, 1.0, 0.0)
    pn = jnp.where((mm & 1) == 1, 1.0, 0.0)
    u2d = -(sims + 3.0 * pn + 12.0 * pos)
    for r in range(_L):
        u_ref[:, r * _P:(r + 1) * _P] = u2d[r:r + 1, :]
    lane = jax.lax.broadcasted_iota(jnp.int32, (1, 128), 1)
    kpos = jnp.sum(pos).astype(jnp.int32)
    kpn = jnp.sum(pn).astype(jnp.int32)
    cnt_ref[...] = jnp.where(lane == 0, kpos, jnp.where(lane == 1, kpn, 0))


def _main_body(k_ref, u_row_ref, out_ref):
    nb = (k_ref[0, 0] + (_BI - 1)) // _BI
    kpos_f = k_ref[0, 0].astype(jnp.float32)
    kpn_f = k_ref[0, 1].astype(jnp.float32)

    def body(blk, carry):
        num_acc, npos_acc = carry
        off = pl.multiple_of(blk * _BI, _BI)
        u_row = u_row_ref[:, pl.ds(off, _BI)]
        u_col = jax.lax.transpose(u_row, (1, 0))
        pos_col = jnp.where(u_col < -10.0, 1.0, 0.0)
        pn_col = jnp.where(u_col < -1.5, 1.0, 0.0)
        s_col = -u_col - 3.0 * pn_col - 12.0 * pos_col
        sc50 = (s_col * (0.5 * _INV_T)).astype(jnp.bfloat16)

        accs = []
        for jc in range(_N // _JC):
            v = u_row_ref[:, jc * _JC:(jc + 1) * _JC]
            m_pn = v < -1.5
            m_pp = v < -10.0
            s_j = (-v - jnp.where(m_pn, 3.0, 0.0)
                   - jnp.where(m_pp, 12.0, 0.0))
            sj50 = (s_j * (0.5 * _INV_T)).astype(jnp.bfloat16)
            t = jax.lax.tanh(sj50 - sc50)
            w = jnp.concatenate(
                [jnp.where(m_pn, 1.0, 0.0),
                 jnp.where(m_pp, 1.0, 0.0)],
                axis=0).astype(jnp.bfloat16)
            accs.append(jax.lax.dot_general(
                t, w, (((1,), (1,)), ((), ())),
                preferred_element_type=jnp.float32,
            ))
        while len(accs) > 1:
            accs = [a + b for a, b in zip(accs[::2], accs[1::2])]
        acc = accs[0]
        rpn = 1.0 + 0.5 * (kpn_f + acc[:, 0:1])
        rp = 1.0 + 0.5 * (kpos_f + acc[:, 1:2])
        num_acc = num_acc + pos_col * rp / rpn
        npos_acc = npos_acc + pos_col
        return num_acc, npos_acc

    zero_col = jnp.zeros((_BI, 1), jnp.float32)
    num_v, npos_v = jax.lax.fori_loop(0, nb, body, (zero_col, zero_col))
    num = jnp.sum(num_v)
    npos = jnp.sum(npos_v)
    lane = jax.lax.broadcasted_iota(jnp.int32, (1, 128), 1)
    out_ref[...] = jnp.where(lane == 0, -(num / npos), 0.0)


def kernel(landmark_embeddings, patch_features, pos_patches, pos_neg_patches):
    mm = ((pos_patches.astype(jnp.uint8) << 1)
          | pos_neg_patches.astype(jnp.uint8))

    u, cnt = pl.pallas_call(
        _pack_body,
        out_shape=(
            jax.ShapeDtypeStruct((1, _N), jnp.float32),
            jax.ShapeDtypeStruct((1, 128), jnp.int32),
        ),
    )(landmark_embeddings, patch_features, mm)

    u_sorted = jax.lax.sort(u.reshape(-1), dimension=0).reshape(1, _N)

    out = pl.pallas_call(
        _main_body,
        in_specs=[
            pl.BlockSpec(memory_space=pltpu.SMEM),
            pl.BlockSpec((1, _N), lambda: (0, 0)),
        ],
        out_specs=pl.BlockSpec((1, 128), lambda: (0, 0)),
        out_shape=jax.ShapeDtypeStruct((1, 128), jnp.float32),
    )(cnt, u_sorted)

    return out[0, 0]
